# baseline (speedup 1.0000x reference)
import functools

import jax
import jax.numpy as jnp
from jax import lax
from jax.experimental import pallas as pl
from jax.experimental.pallas import tpu as pltpu
from jax.experimental.pallas import tpu_sc as plsc

VOCAB = 1000000
DIM = 64
BATCH = 16384
HIST = 50

NC = 2
NS = 16
NW = NC * NS

BPW = BATCH // NW
B_PER_W = BPW * HIST
CB = 8
R = CB * HIST
STREAMS = ((0, 128), (128, 128), (256, 128), (384, 16))
CHUNKS = BPW // CB
K = 4
T = CHUNKS // K


@functools.partial(
    pl.kernel,
    out_type=jax.ShapeDtypeStruct((BATCH, HIST, DIM), jnp.float32),
    mesh=plsc.VectorSubcoreMesh(core_axis_name="c", subcore_axis_name="s"),
    scratch_types=[
        pltpu.VMEM((B_PER_W,), jnp.int32),
        [pltpu.VMEM((R, DIM), jnp.float32) for _ in range(K)],
        [pltpu.SemaphoreType.DMA for _ in range(K)],
        [pltpu.SemaphoreType.DMA for _ in range(K)],
    ],
    compiler_params=pltpu.CompilerParams(use_tc_tiling_on_sc=False),
)
def _gather_kernel(idx_hbm, table_hbm, out_hbm, idx_v, gbuf, sem_g, sem_o):
    wid = lax.axis_index("s") * NC + lax.axis_index("c")
    base_i = wid * B_PER_W
    base_b = wid * BPW

    def fire_gathers(b, c):
        return [
            pltpu.async_copy(
                table_hbm.at[idx_v.at[pl.ds(c * R + so, sl)]],
                gbuf[b].at[pl.ds(so, sl)],
                sem_g[b],
            )
            for so, sl in STREAMS
        ]

    def fire_outs(b, c):
        for i in range(CB):
            pltpu.async_copy(
                gbuf[b].at[pl.ds(i * HIST, HIST)],
                out_hbm.at[base_b + c * CB + i],
                sem_o[b],
            )

    def drain_outs(b, c):
        for i in range(CB):
            pltpu.make_async_copy(
                gbuf[b].at[pl.ds(i * HIST, HIST)],
                out_hbm.at[base_b + c * CB + i],
                sem_o[b],
            ).wait()

    pltpu.sync_copy(idx_hbm.at[pl.ds(base_i, B_PER_W)], idx_v)

    ds0 = [fire_gathers(b, b) for b in range(K)]
    for b in range(K):
        for d in ds0[b]:
            d.wait()
        fire_outs(b, b)

    def body(t, _):
        c0 = t * K
        ds = []
        for b in range(K):
            drain_outs(b, c0 - K + b)
            ds.append(fire_gathers(b, c0 + b))
        for b in range(K):
            for d in ds[b]:
                d.wait()
            fire_outs(b, c0 + b)
        return 0

    lax.fori_loop(1, T, body, 0)

    for b in range(K):
        drain_outs(b, (T - 1) * K + b)


def kernel(x, weight):
    idx = x.reshape(BATCH * HIST).astype(jnp.int32)
    return _gather_kernel(idx, weight)
